# Initial kernel scaffold; baseline (speedup 1.0000x reference)
import jax
import jax.numpy as jnp
from jax.experimental import pallas as pl


def kernel(x_paper, x_author, edge_index_cites, edge_index_writes, edge_index_rev_writes, neighbor_mask_node_paper, neighbor_mask_node_author, neighbor_mask_edge_cites, neighbor_mask_edge_writes, neighbor_mask_edge_rev_writes, Wl, Wr, b):
    raise NotImplementedError("write your pallas kernel here")


# speedup vs baseline: 1.0243x; 1.0243x over previous
import functools

import jax
import jax.numpy as jnp
from jax import lax
from jax.experimental import pallas as pl
from jax.experimental.pallas import tpu as pltpu

N_P = 50000
N_A = 25000
NE = 200000
D = 128


def _dense2_body(s0, c0, s1, c1, x, wl0, wl1, wr, bias, nm, o):
    m0 = s0[...] * (1.0 / jnp.maximum(c0[...], 1.0))
    m1 = s1[...] * (1.0 / jnp.maximum(c1[...], 1.0))
    acc = jnp.dot(m0, wl0[...], preferred_element_type=jnp.float32)
    acc += jnp.dot(m1, wl1[...], preferred_element_type=jnp.float32)
    acc += jnp.dot(x[...], wr[...], preferred_element_type=jnp.float32)
    acc += bias[...]
    acc = jnp.maximum(acc, 0.0)
    if nm is not None:
        acc *= (nm[...] <= 1).astype(jnp.float32)
    o[...] = acc


def _dense1_body(s0, c0, x, wl0, wr, bias, nm, o):
    m0 = s0[...] * (1.0 / jnp.maximum(c0[...], 1.0))
    acc = jnp.dot(m0, wl0[...], preferred_element_type=jnp.float32)
    acc += jnp.dot(x[...], wr[...], preferred_element_type=jnp.float32)
    acc += bias[...]
    acc = jnp.maximum(acc, 0.0)
    if nm is not None:
        acc *= (nm[...] <= 1).astype(jnp.float32)
    o[...] = acc


def _row_spec(blk):
    return pl.BlockSpec((blk, D), lambda i: (i, 0))


def _cnt_spec(blk):
    return pl.BlockSpec((blk, 1), lambda i: (i, 0))


def _w_spec():
    return pl.BlockSpec((D, D), lambda i: (0, 0))


def _b_spec():
    return pl.BlockSpec((1, D), lambda i: (0, 0))


@functools.partial(jax.jit, static_argnames=("n", "masked"))
def _dense2(s0, c0, s1, c1, x, wl0, wl1, wr, bias, nm, *, n, masked):
    blk = 2048
    grid = (pl.cdiv(n, blk),)
    in_specs = [_row_spec(blk), _cnt_spec(blk), _row_spec(blk), _cnt_spec(blk),
                _row_spec(blk), _w_spec(), _w_spec(), _w_spec(), _b_spec()]
    args = [s0, c0, s1, c1, x, wl0, wl1, wr, bias]
    if masked:
        in_specs.append(_cnt_spec(blk))
        args.append(nm)
        body = _dense2_body
    else:
        body = functools.partial(_dense2_body, nm=None)

        def body(*a):
            _dense2_body(*a[:9], None, a[9])
    return pl.pallas_call(
        body,
        grid=grid,
        in_specs=in_specs,
        out_specs=_row_spec(blk),
        out_shape=jax.ShapeDtypeStruct((n, D), jnp.float32),
    )(*args)


@functools.partial(jax.jit, static_argnames=("n", "masked"))
def _dense1(s0, c0, x, wl0, wr, bias, nm, *, n, masked):
    blk = 2048
    grid = (pl.cdiv(n, blk),)
    in_specs = [_row_spec(blk), _cnt_spec(blk), _row_spec(blk),
                _w_spec(), _w_spec(), _b_spec()]
    args = [s0, c0, x, wl0, wr, bias]
    if masked:
        in_specs.append(_cnt_spec(blk))
        args.append(nm)
        body = _dense1_body
    else:

        def body(*a):
            _dense1_body(*a[:6], None, a[6])
    return pl.pallas_call(
        body,
        grid=grid,
        in_specs=in_specs,
        out_specs=_row_spec(blk),
        out_shape=jax.ShapeDtypeStruct((n, D), jnp.float32),
    )(*args)


def _seg(x_src, src, dst, val, n_dst):
    msg = jnp.take(x_src, src, axis=0)
    if val is not None:
        vf = val.astype(jnp.float32)
        msg = msg * vf[:, None]
    else:
        vf = jnp.ones((src.shape[0],), jnp.float32)
    s = jax.ops.segment_sum(msg, dst, num_segments=n_dst)
    c = jax.ops.segment_sum(vf, dst, num_segments=n_dst)
    return s, c[:, None]


def kernel(x_paper, x_author, edge_index_cites, edge_index_writes,
           edge_index_rev_writes, neighbor_mask_node_paper,
           neighbor_mask_node_author, neighbor_mask_edge_cites,
           neighbor_mask_edge_writes, neighbor_mask_edge_rev_writes,
           Wl, Wr, b):
    xp, xa = x_paper, x_author
    nmp = neighbor_mask_node_paper.astype(jnp.int32)[:, None]
    nma = neighbor_mask_node_author.astype(jnp.int32)[:, None]
    ec = edge_index_cites.astype(jnp.int32)
    ew = edge_index_writes.astype(jnp.int32)
    er = edge_index_rev_writes.astype(jnp.int32)
    for i in range(2):
        layer = 2 - i
        if layer == 2:
            evc = evw = evr = None
            masked = False
        else:
            evc = neighbor_mask_edge_cites == 0
            evw = neighbor_mask_edge_writes == 0
            evr = neighbor_mask_edge_rev_writes == 0
            masked = True
        s_c, c_c = _seg(xp, ec[0], ec[1], evc, N_P)
        s_w, c_w = _seg(xa, ew[0], ew[1], evw, N_P)
        s_r, c_r = _seg(xp, er[0], er[1], evr, N_A)
        bias_p = (b[i, 0] + b[i, 1])[None, :]
        bias_a = b[i, 2][None, :]
        wr_p = Wr[i, 0] + Wr[i, 1]
        new_p = _dense2(s_c, c_c, s_w, c_w, xp, Wl[i, 0], Wl[i, 1], wr_p,
                        bias_p, nmp, n=N_P, masked=masked)
        new_a = _dense1(s_r, c_r, xa, Wl[i, 2], Wr[i, 2], bias_a, nma,
                        n=N_A, masked=masked)
        xp, xa = new_p, new_a
    return xp, xa


# speedup vs baseline: 1.7708x; 1.7289x over previous
import functools

import jax
import jax.numpy as jnp
from jax import lax
from jax.experimental import pallas as pl
from jax.experimental.pallas import tpu as pltpu

N_P = 50000
N_A = 25000
NE = 200000
D = 128


def _dense2_body(s0, c0, s1, c1, x, wl0, wl1, wr, bias, nm, o):
    m0 = s0[...] * (1.0 / jnp.maximum(c0[...], 1.0))
    m1 = s1[...] * (1.0 / jnp.maximum(c1[...], 1.0))
    acc = jnp.dot(m0, wl0[...], preferred_element_type=jnp.float32)
    acc += jnp.dot(m1, wl1[...], preferred_element_type=jnp.float32)
    acc += jnp.dot(x[...], wr[...], preferred_element_type=jnp.float32)
    acc += bias[...]
    acc = jnp.maximum(acc, 0.0)
    if nm is not None:
        acc *= (nm[...] <= 1).astype(jnp.float32)
    o[...] = acc


def _dense1_body(s0, c0, x, wl0, wr, bias, nm, o):
    m0 = s0[...] * (1.0 / jnp.maximum(c0[...], 1.0))
    acc = jnp.dot(m0, wl0[...], preferred_element_type=jnp.float32)
    acc += jnp.dot(x[...], wr[...], preferred_element_type=jnp.float32)
    acc += bias[...]
    acc = jnp.maximum(acc, 0.0)
    if nm is not None:
        acc *= (nm[...] <= 1).astype(jnp.float32)
    o[...] = acc


def _row_spec(blk):
    return pl.BlockSpec((blk, D), lambda i: (i, 0))


def _cnt_spec(blk):
    return pl.BlockSpec((blk, 1), lambda i: (i, 0))


def _w_spec():
    return pl.BlockSpec((D, D), lambda i: (0, 0))


def _b_spec():
    return pl.BlockSpec((1, D), lambda i: (0, 0))


@functools.partial(jax.jit, static_argnames=("n", "masked"))
def _dense2(s0, c0, s1, c1, x, wl0, wl1, wr, bias, nm, *, n, masked):
    blk = 2048
    grid = (pl.cdiv(n, blk),)
    in_specs = [_row_spec(blk), _cnt_spec(blk), _row_spec(blk), _cnt_spec(blk),
                _row_spec(blk), _w_spec(), _w_spec(), _w_spec(), _b_spec()]
    args = [s0, c0, s1, c1, x, wl0, wl1, wr, bias]
    if masked:
        in_specs.append(_cnt_spec(blk))
        args.append(nm)
        body = _dense2_body
    else:
        body = functools.partial(_dense2_body, nm=None)

        def body(*a):
            _dense2_body(*a[:9], None, a[9])
    return pl.pallas_call(
        body,
        grid=grid,
        in_specs=in_specs,
        out_specs=_row_spec(blk),
        out_shape=jax.ShapeDtypeStruct((n, D), jnp.float32),
    )(*args)


@functools.partial(jax.jit, static_argnames=("n", "masked"))
def _dense1(s0, c0, x, wl0, wr, bias, nm, *, n, masked):
    blk = 2048
    grid = (pl.cdiv(n, blk),)
    in_specs = [_row_spec(blk), _cnt_spec(blk), _row_spec(blk),
                _w_spec(), _w_spec(), _b_spec()]
    args = [s0, c0, x, wl0, wr, bias]
    if masked:
        in_specs.append(_cnt_spec(blk))
        args.append(nm)
        body = _dense1_body
    else:

        def body(*a):
            _dense1_body(*a[:6], None, a[6])
    return pl.pallas_call(
        body,
        grid=grid,
        in_specs=in_specs,
        out_specs=_row_spec(blk),
        out_shape=jax.ShapeDtypeStruct((n, D), jnp.float32),
    )(*args)


from jax.experimental.pallas import tpu_sc as plsc

NC = 2
NS = 16
LANES = 16
EPAD = 200704
EP = EPAD // NS
KB = 128
ZR = 16


def _seg_body(cpc, CH, refs):
    FR = CH // NS
    CW = CH + KB
    (xsrc, esrc, edst, s_out, cnt_out,
     e_src, e_dst, srcc, rows, zbuf, srcrow, dstrow,
     cntl, cntbuf, cntsum, acc, cntsh) = refs
    c = lax.axis_index("c")
    s = lax.axis_index("s")
    zv = jnp.zeros((LANES,), jnp.float32)
    ones = jnp.ones((LANES,), jnp.float32)

    ebase = s * EP
    pltpu.sync_copy(esrc.at[pl.ds(ebase, EP)], e_src)
    pltpu.sync_copy(edst.at[pl.ds(ebase, EP)], e_dst)

    def zb_body(r, _):
        for j in range(8):
            zbuf[r, pl.ds(j * LANES, LANES)] = zv
        return 0
    lax.fori_loop(0, ZR, zb_body, 0)

    for k in range(cpc):
        lo = (NC * k + c) * CH if cpc > 1 else c * CH
        for r in range(FR // ZR):
            pltpu.sync_copy(zbuf, acc.at[pl.ds(s * FR + r * ZR, ZR)])

        def cz_body(t, _):
            cntl[pl.ds(t * LANES, LANES)] = zv
            return 0
        lax.fori_loop(0, CW // LANES, cz_body, 0)
        plsc.subcore_barrier()

        lane = lax.iota(jnp.int32, LANES)

        def scan_body(t, p):
            sl = pl.ds(t * LANES, LANES)
            d16 = e_dst[sl]
            s16 = e_src[sl]
            m = (d16 >= lo) & (d16 < lo + CH)
            dl = d16 - lo
            key = jnp.where(m, lane, lane + LANES)
            pk = s16 | (dl << 16)
            _, sv = plsc.sort_key_val(key, pk)
            srcc[pl.ds(p, LANES)] = sv
            dls = jnp.where(m, dl, CH)
            plsc.addupdate_scatter(cntl, [dls], ones, mask=m)
            return p + plsc.all_reduce_population_count(m)[0]

        p = lax.fori_loop(0, EP // LANES, scan_body, jnp.int32(0))

        pad_pk = jnp.full((LANES,), CH << 16, jnp.int32)
        for j in range(KB // LANES):
            srcc[pl.ds(p + j * LANES, LANES)] = pad_pk

        def batch_body(bi, _):
            off = bi * KB
            for j in range(KB // LANES):
                v = srcc[pl.ds(off + j * LANES, LANES)]
                srcrow[0, pl.ds(j * LANES, LANES)] = v & 0xFFFF
                dstrow[0, pl.ds(j * LANES, LANES)] = v >> 16
            pltpu.sync_copy(xsrc.at[srcrow.at[0]], rows)
            pltpu.sync_copy(rows, acc.at[dstrow.at[0]], add=True)
            return 0

        nb = (p + KB - 1) // KB
        lax.fori_loop(0, nb, batch_body, 0)
        plsc.subcore_barrier()

        pltpu.sync_copy(acc.at[pl.ds(s * FR, FR)],
                        s_out.at[pl.ds(lo + s * FR, FR)])
        pltpu.sync_copy(cntl, cntsh.at[pl.ds(s * CW, CW)])
        plsc.subcore_barrier()
        for r in range(NS):
            pltpu.sync_copy(cntsh.at[pl.ds(r * CW + s * FR, FR)],
                            cntbuf.at[pl.ds(r * FR, FR)])

        def csum_body(t, _):
            sl = pl.ds(t * LANES, LANES)
            v = cntbuf[pl.ds(t * LANES, LANES)]
            for r in range(1, NS):
                v = v + cntbuf[pl.ds(r * FR + t * LANES, LANES)]
            cntsum[sl] = v
            return 0
        lax.fori_loop(0, FR // LANES, csum_body, 0)
        pltpu.sync_copy(cntsum, cnt_out.at[pl.ds(lo + s * FR, FR)])
        plsc.subcore_barrier()


def _make_seg_sc(n_pad, CH):
    FR = CH // NS
    CW = CH + KB
    cpc = n_pad // (CH * NC)
    mesh = plsc.VectorSubcoreMesh(core_axis_name="c", subcore_axis_name="s",
                                  num_cores=NC, num_subcores=NS)
    scratch = [
        pltpu.VMEM((EP,), jnp.int32),
        pltpu.VMEM((EP,), jnp.int32),
    ]
    scratch += [
        pltpu.VMEM((EP + KB,), jnp.int32),
        pltpu.VMEM((KB, D), jnp.float32),
        pltpu.VMEM((ZR, D), jnp.float32),
        pltpu.VMEM((1, KB), jnp.int32),
        pltpu.VMEM((1, KB), jnp.int32),
        pltpu.VMEM((CW,), jnp.float32),
        pltpu.VMEM((NS * FR,), jnp.float32),
        pltpu.VMEM((FR,), jnp.float32),
        pltpu.VMEM_SHARED((CH + LANES, D), jnp.float32),
        pltpu.VMEM_SHARED((NS * CW,), jnp.float32),
    ]
    out_type = [jax.ShapeDtypeStruct((n_pad, D), jnp.float32),
                jax.ShapeDtypeStruct((n_pad,), jnp.float32)]

    def body(*refs):
        _seg_body(cpc, CH, refs)

    return pl.kernel(body, out_type=out_type, mesh=mesh,
                     scratch_types=scratch,
                     compiler_params=pltpu.CompilerParams(
                         needs_layout_passes=False))


NP_PAD = 55296
NA_PAD = 25600
_seg_p = _make_seg_sc(NP_PAD, 4608)
_seg_a = _make_seg_sc(NA_PAD, 2560)


def _pad_edges(e, sentinel):
    npad = EPAD - NE
    src = jnp.concatenate([e[0], jnp.zeros((npad,), jnp.int32)])
    dst = jnp.concatenate([e[1], jnp.full((npad,), sentinel, jnp.int32)])
    return src, dst


def kernel(x_paper, x_author, edge_index_cites, edge_index_writes,
           edge_index_rev_writes, neighbor_mask_node_paper,
           neighbor_mask_node_author, neighbor_mask_edge_cites,
           neighbor_mask_edge_writes, neighbor_mask_edge_rev_writes,
           Wl, Wr, b):
    xp, xa = x_paper, x_author
    nmp = neighbor_mask_node_paper.astype(jnp.int32)[:, None]
    nma = neighbor_mask_node_author.astype(jnp.int32)[:, None]
    ec = edge_index_cites.astype(jnp.int32)
    ew = edge_index_writes.astype(jnp.int32)
    er = edge_index_rev_writes.astype(jnp.int32)
    ec_s, ec_d = _pad_edges(ec, NP_PAD)
    ew_s, ew_d = _pad_edges(ew, NP_PAD)
    er_s, er_d = _pad_edges(er, NA_PAD)
    ec_d1 = jnp.where(jnp.pad(neighbor_mask_edge_cites.astype(jnp.int32),
                              (0, EPAD - NE), constant_values=1) == 0,
                      ec_d, NP_PAD)
    ew_d1 = jnp.where(jnp.pad(neighbor_mask_edge_writes.astype(jnp.int32),
                              (0, EPAD - NE), constant_values=1) == 0,
                      ew_d, NP_PAD)
    er_d1 = jnp.where(jnp.pad(neighbor_mask_edge_rev_writes.astype(jnp.int32),
                              (0, EPAD - NE), constant_values=1) == 0,
                      er_d, NA_PAD)
    for i in range(2):
        layer = 2 - i
        if layer == 2:
            masked = False
            s_c, c_c = _seg_p(xp, ec_s, ec_d)
            s_w, c_w = _seg_p(xa, ew_s, ew_d)
            s_r, c_r = _seg_a(xp, er_s, er_d)
        else:
            masked = True
            s_c, c_c = _seg_p(xp, ec_s, ec_d1)
            s_w, c_w = _seg_p(xa, ew_s, ew_d1)
            s_r, c_r = _seg_a(xp, er_s, er_d1)
        c_c = c_c[:, None]
        c_w = c_w[:, None]
        c_r = c_r[:, None]
        bias_p = (b[i, 0] + b[i, 1])[None, :]
        bias_a = b[i, 2][None, :]
        wr_p = Wr[i, 0] + Wr[i, 1]
        new_p = _dense2(s_c, c_c, s_w, c_w, xp, Wl[i, 0], Wl[i, 1], wr_p,
                        bias_p, nmp, n=N_P, masked=masked)
        new_a = _dense1(s_r, c_r, xa, Wl[i, 2], Wr[i, 2], bias_a, nma,
                        n=N_A, masked=masked)
        xp, xa = new_p, new_a
    return xp, xa


# speedup vs baseline: 2.3988x; 1.3547x over previous
import functools

import jax
import jax.numpy as jnp
from jax import lax
from jax.experimental import pallas as pl
from jax.experimental.pallas import tpu as pltpu

N_P = 50000
N_A = 25000
NE = 200000
D = 128


def _dense2_body(s0, c0, s1, c1, x, wl0, wl1, wr, bias, nm, o):
    m0 = s0[...] * (1.0 / jnp.maximum(c0[...], 1.0))
    m1 = s1[...] * (1.0 / jnp.maximum(c1[...], 1.0))
    acc = jnp.dot(m0, wl0[...], preferred_element_type=jnp.float32)
    acc += jnp.dot(m1, wl1[...], preferred_element_type=jnp.float32)
    acc += jnp.dot(x[...], wr[...], preferred_element_type=jnp.float32)
    acc += bias[...]
    acc = jnp.maximum(acc, 0.0)
    if nm is not None:
        acc *= (nm[...] <= 1).astype(jnp.float32)
    o[...] = acc


def _dense1_body(s0, c0, x, wl0, wr, bias, nm, o):
    m0 = s0[...] * (1.0 / jnp.maximum(c0[...], 1.0))
    acc = jnp.dot(m0, wl0[...], preferred_element_type=jnp.float32)
    acc += jnp.dot(x[...], wr[...], preferred_element_type=jnp.float32)
    acc += bias[...]
    acc = jnp.maximum(acc, 0.0)
    if nm is not None:
        acc *= (nm[...] <= 1).astype(jnp.float32)
    o[...] = acc


def _row_spec(blk):
    return pl.BlockSpec((blk, D), lambda i: (i, 0))


def _cnt_spec(blk):
    return pl.BlockSpec((blk, 1), lambda i: (i, 0))


def _w_spec():
    return pl.BlockSpec((D, D), lambda i: (0, 0))


def _b_spec():
    return pl.BlockSpec((1, D), lambda i: (0, 0))


@functools.partial(jax.jit, static_argnames=("n", "masked"))
def _dense2(s0, c0, s1, c1, x, wl0, wl1, wr, bias, nm, *, n, masked):
    blk = 2048
    grid = (pl.cdiv(n, blk),)
    in_specs = [_row_spec(blk), _cnt_spec(blk), _row_spec(blk), _cnt_spec(blk),
                _row_spec(blk), _w_spec(), _w_spec(), _w_spec(), _b_spec()]
    args = [s0, c0, s1, c1, x, wl0, wl1, wr, bias]
    if masked:
        in_specs.append(_cnt_spec(blk))
        args.append(nm)
        body = _dense2_body
    else:
        body = functools.partial(_dense2_body, nm=None)

        def body(*a):
            _dense2_body(*a[:9], None, a[9])
    return pl.pallas_call(
        body,
        grid=grid,
        in_specs=in_specs,
        out_specs=_row_spec(blk),
        out_shape=jax.ShapeDtypeStruct((n, D), jnp.float32),
    )(*args)


@functools.partial(jax.jit, static_argnames=("n", "masked"))
def _dense1(s0, c0, x, wl0, wr, bias, nm, *, n, masked):
    blk = 2048
    grid = (pl.cdiv(n, blk),)
    in_specs = [_row_spec(blk), _cnt_spec(blk), _row_spec(blk),
                _w_spec(), _w_spec(), _b_spec()]
    args = [s0, c0, x, wl0, wr, bias]
    if masked:
        in_specs.append(_cnt_spec(blk))
        args.append(nm)
        body = _dense1_body
    else:

        def body(*a):
            _dense1_body(*a[:6], None, a[6])
    return pl.pallas_call(
        body,
        grid=grid,
        in_specs=in_specs,
        out_specs=_row_spec(blk),
        out_shape=jax.ShapeDtypeStruct((n, D), jnp.float32),
    )(*args)


from jax.experimental.pallas import tpu_sc as plsc

NC = 2
NS = 16
LANES = 16
EPAD = 200704
EP = EPAD // NS
KB = 128
ZR = 16


def _seg_body(cpc, CH, refs):
    FR = CH // NS
    CW = CH + KB
    (xsrc, esrc, edst, s_out, cnt_out,
     e_src, e_dst, srcc, rows, zbuf, srcrow, dstrow,
     cntl, cntbuf, cntsum, acc, cntsh) = refs
    c = lax.axis_index("c")
    s = lax.axis_index("s")
    zv = jnp.zeros((LANES,), jnp.float32)
    ones = jnp.ones((LANES,), jnp.float32)

    ebase = s * EP
    pltpu.sync_copy(esrc.at[pl.ds(ebase, EP)], e_src)
    pltpu.sync_copy(edst.at[pl.ds(ebase, EP)], e_dst)

    def zb_body(r, _):
        for j in range(8):
            zbuf[r, pl.ds(j * LANES, LANES)] = zv
        return 0
    lax.fori_loop(0, ZR, zb_body, 0)

    for k in range(cpc):
        lo = (NC * k + c) * CH if cpc > 1 else c * CH
        for r in range(FR // ZR):
            pltpu.sync_copy(zbuf, acc.at[pl.ds(s * FR + r * ZR, ZR)])

        def cz_body(t, _):
            cntl[pl.ds(t * LANES, LANES)] = zv
            return 0
        lax.fori_loop(0, CW // LANES, cz_body, 0)
        plsc.subcore_barrier()

        lane = lax.iota(jnp.int32, LANES)

        def scan_body(t, p):
            sl = pl.ds(t * LANES, LANES)
            d16 = e_dst[sl]
            s16 = e_src[sl]
            m = (d16 >= lo) & (d16 < lo + CH)
            dl = d16 - lo
            key = jnp.where(m, lane, lane + LANES)
            pk = s16 | (dl << 16)
            _, sv = plsc.sort_key_val(key, pk)
            srcc[pl.ds(p, LANES)] = sv
            dls = jnp.where(m, dl, CH)
            plsc.addupdate_scatter(cntl, [dls], ones, mask=m)
            return p + plsc.all_reduce_population_count(m)[0]

        p = lax.fori_loop(0, EP // LANES, scan_body, jnp.int32(0))

        pad_pk = jnp.full((LANES,), CH << 16, jnp.int32)
        for j in range(KB // LANES):
            srcc[pl.ds(p + j * LANES, LANES)] = pad_pk

        def batch_body(bi, _):
            off = bi * KB
            for j in range(KB // LANES):
                v = srcc[pl.ds(off + j * LANES, LANES)]
                srcrow[0, pl.ds(j * LANES, LANES)] = v & 0xFFFF
                dstrow[0, pl.ds(j * LANES, LANES)] = v >> 16
            pltpu.sync_copy(xsrc.at[srcrow.at[0]], rows)
            pltpu.sync_copy(rows, acc.at[dstrow.at[0]], add=True)
            return 0

        nb = (p + KB - 1) // KB
        lax.fori_loop(0, nb, batch_body, 0)
        plsc.subcore_barrier()

        pltpu.sync_copy(acc.at[pl.ds(s * FR, FR)],
                        s_out.at[pl.ds(lo + s * FR, FR)])
        pltpu.sync_copy(cntl, cntsh.at[pl.ds(s * CW, CW)])
        plsc.subcore_barrier()
        for r in range(NS):
            pltpu.sync_copy(cntsh.at[pl.ds(r * CW + s * FR, FR)],
                            cntbuf.at[pl.ds(r * FR, FR)])

        def csum_body(t, _):
            sl = pl.ds(t * LANES, LANES)
            v = cntbuf[pl.ds(t * LANES, LANES)]
            for r in range(1, NS):
                v = v + cntbuf[pl.ds(r * FR + t * LANES, LANES)]
            cntsum[sl] = v
            return 0
        lax.fori_loop(0, FR // LANES, csum_body, 0)
        pltpu.sync_copy(cntsum, cnt_out.at[pl.ds(lo + s * FR, FR)])
        plsc.subcore_barrier()


def _make_seg_sc(n_pad, CH):
    FR = CH // NS
    CW = CH + KB
    cpc = n_pad // (CH * NC)
    mesh = plsc.VectorSubcoreMesh(core_axis_name="c", subcore_axis_name="s",
                                  num_cores=NC, num_subcores=NS)
    scratch = [
        pltpu.VMEM((EP,), jnp.int32),
        pltpu.VMEM((EP,), jnp.int32),
    ]
    scratch += [
        pltpu.VMEM((EP + KB,), jnp.int32),
        pltpu.VMEM((KB, D), jnp.float32),
        pltpu.VMEM((ZR, D), jnp.float32),
        pltpu.VMEM((1, KB), jnp.int32),
        pltpu.VMEM((1, KB), jnp.int32),
        pltpu.VMEM((CW,), jnp.float32),
        pltpu.VMEM((NS * FR,), jnp.float32),
        pltpu.VMEM((FR,), jnp.float32),
        pltpu.VMEM_SHARED((CH + LANES, D), jnp.float32),
        pltpu.VMEM_SHARED((NS * CW,), jnp.float32),
    ]
    out_type = [jax.ShapeDtypeStruct((n_pad, D), jnp.float32),
                jax.ShapeDtypeStruct((n_pad,), jnp.float32)]

    def body(*refs):
        _seg_body(cpc, CH, refs)

    return pl.kernel(body, out_type=out_type, mesh=mesh,
                     scratch_types=scratch,
                     compiler_params=pltpu.CompilerParams(
                         needs_layout_passes=False))


NP_PAD = 51200
NA_PAD = 51200
_seg_p = _make_seg_sc(NP_PAD, 6400)
_seg_a = _seg_p


def _pad_edges(e, sentinel):
    npad = EPAD - NE
    src = jnp.concatenate([e[0], jnp.zeros((npad,), jnp.int32)])
    dst = jnp.concatenate([e[1], jnp.full((npad,), sentinel, jnp.int32)])
    return src, dst


def kernel(x_paper, x_author, edge_index_cites, edge_index_writes,
           edge_index_rev_writes, neighbor_mask_node_paper,
           neighbor_mask_node_author, neighbor_mask_edge_cites,
           neighbor_mask_edge_writes, neighbor_mask_edge_rev_writes,
           Wl, Wr, b):
    xp, xa = x_paper, x_author
    nmp = neighbor_mask_node_paper.astype(jnp.int32)[:, None]
    nma = neighbor_mask_node_author.astype(jnp.int32)[:, None]
    ec = edge_index_cites.astype(jnp.int32)
    ew = edge_index_writes.astype(jnp.int32)
    er = edge_index_rev_writes.astype(jnp.int32)
    ec_s, ec_d = _pad_edges(ec, NP_PAD)
    ew_s, ew_d = _pad_edges(ew, NP_PAD)
    er_s, er_d = _pad_edges(er, NA_PAD)
    ec_d1 = jnp.where(jnp.pad(neighbor_mask_edge_cites.astype(jnp.int32),
                              (0, EPAD - NE), constant_values=1) == 0,
                      ec_d, NP_PAD)
    ew_d1 = jnp.where(jnp.pad(neighbor_mask_edge_writes.astype(jnp.int32),
                              (0, EPAD - NE), constant_values=1) == 0,
                      ew_d, NP_PAD)
    er_d1 = jnp.where(jnp.pad(neighbor_mask_edge_rev_writes.astype(jnp.int32),
                              (0, EPAD - NE), constant_values=1) == 0,
                      er_d, NA_PAD)
    for i in range(2):
        layer = 2 - i
        xa_pad = jnp.concatenate(
            [xa, jnp.zeros((N_P - N_A, D), jnp.float32)])
        if layer == 2:
            masked = False
            s_c, c_c = _seg_p(xp, ec_s, ec_d)
            s_w, c_w = _seg_p(xa_pad, ew_s, ew_d)
            s_r, c_r = _seg_a(xp, er_s, er_d)
        else:
            masked = True
            s_c, c_c = _seg_p(xp, ec_s, ec_d1)
            s_w, c_w = _seg_p(xa_pad, ew_s, ew_d1)
            s_r, c_r = _seg_a(xp, er_s, er_d1)
        c_c = c_c[:, None]
        c_w = c_w[:, None]
        c_r = c_r[:, None]
        bias_p = (b[i, 0] + b[i, 1])[None, :]
        bias_a = b[i, 2][None, :]
        wr_p = Wr[i, 0] + Wr[i, 1]
        new_p = _dense2(s_c, c_c, s_w, c_w, xp, Wl[i, 0], Wl[i, 1], wr_p,
                        bias_p, nmp, n=N_P, masked=masked)
        new_a = _dense1(s_r, c_r, xa, Wl[i, 2], Wr[i, 2], bias_a, nma,
                        n=N_A, masked=masked)
        xp, xa = new_p, new_a
    return xp, xa
